# baseline (speedup 1.0000x reference)
import jax
import jax.numpy as jnp
from jax import lax
from jax.experimental import pallas as pl
from jax.experimental.pallas import tpu as pltpu
from jax.experimental.pallas import tpu_sc as plsc

_BATCH = 4
_SEQ = 8192
_D = 768
_NC = 2
_NS = 16
_NW = _NC * _NS
_ROWS_PER_W = _SEQ // _NW
_CHUNK = 64
_NCHUNK = _ROWS_PER_W // _CHUNK
_PORT_BATCHES = (0, 1)
_H2H_BATCHES = (2, 3)


def _sc_body(pe_hbm, out_hbm, buf0, buf1, isem0, isem1, osem0, osem1, hsem):
    wid = lax.axis_index("s") * _NC + lax.axis_index("c")
    base = wid * _ROWS_PER_W
    bufs = (buf0, buf1)
    isems = (isem0, isem1)
    osems = (osem0, osem1)

    slab = pl.ds(base, _ROWS_PER_W)
    h2h = [
        pltpu.make_async_copy(pe_hbm.at[slab, :], out_hbm.at[b, slab, :], hsem)
        for b in _H2H_BATCHES
    ]
    for cp in h2h:
        cp.start()

    def in_copy(k):
        return pltpu.make_async_copy(
            pe_hbm.at[pl.ds(base + k * _CHUNK, _CHUNK), :],
            bufs[k % 2],
            isems[k % 2],
        )

    def out_copies(k):
        return [
            pltpu.make_async_copy(
                bufs[k % 2],
                out_hbm.at[b, pl.ds(base + k * _CHUNK, _CHUNK), :],
                osems[k % 2],
            )
            for b in _PORT_BATCHES
        ]

    in_copy(0).start()
    in_copy(1).start()
    for k in range(_NCHUNK):
        if k >= 2:
            for cp in out_copies(k - 2):
                cp.wait()
            in_copy(k).start()
        in_copy(k).wait()
        for cp in out_copies(k):
            cp.start()
    for k in (_NCHUNK - 2, _NCHUNK - 1):
        for cp in out_copies(k):
            cp.wait()
    for cp in h2h:
        cp.wait()


def kernel(input_ids, pos_embed):
    batch, seq_len = input_ids.shape
    d = pos_embed.shape[1]
    mesh = plsc.VectorSubcoreMesh(core_axis_name="c", subcore_axis_name="s")
    sc_call = pl.kernel(
        _sc_body,
        out_type=jax.ShapeDtypeStruct((batch, seq_len, d), jnp.float32),
        mesh=mesh,
        scratch_types=[
            pltpu.VMEM((_CHUNK, _D), jnp.float32),
            pltpu.VMEM((_CHUNK, _D), jnp.float32),
            pltpu.SemaphoreType.DMA,
            pltpu.SemaphoreType.DMA,
            pltpu.SemaphoreType.DMA,
            pltpu.SemaphoreType.DMA,
            pltpu.SemaphoreType.DMA,
        ],
    )
    return sc_call(pos_embed[:seq_len])


# speedup vs baseline: 25.8854x; 25.8854x over previous
import jax
import jax.numpy as jnp
from jax import lax
from jax.experimental import pallas as pl
from jax.experimental.pallas import tpu as pltpu
from jax.experimental.pallas import tpu_sc as plsc

_BATCH = 4
_SEQ = 8192
_D = 768
_NC = 2
_NS = 16
_NW = _NC * _NS
_ROWS_PER_W = _SEQ // _NW
_CHUNK = 64
_NCHUNK = _ROWS_PER_W // _CHUNK


def _sc_body(pe_hbm, out_hbm, buf0, buf1, isem0, isem1, osem0, osem1):
    wid = lax.axis_index("s") * _NC + lax.axis_index("c")
    base = wid * _ROWS_PER_W
    bufs = (buf0, buf1)
    isems = (isem0, isem1)
    osems = (osem0, osem1)

    def in_copy(k):
        return pltpu.make_async_copy(
            pe_hbm.at[pl.ds(base + k * _CHUNK, _CHUNK), :],
            bufs[k % 2],
            isems[k % 2],
        )

    def out_copies(k):
        return [
            pltpu.make_async_copy(
                bufs[k % 2],
                out_hbm.at[b, pl.ds(base + k * _CHUNK, _CHUNK), :],
                osems[k % 2],
            )
            for b in range(_BATCH)
        ]

    in_copy(0).start()
    in_copy(1).start()
    for k in range(_NCHUNK):
        if k >= 2:
            for cp in out_copies(k - 2):
                cp.wait()
            in_copy(k).start()
        in_copy(k).wait()
        for cp in out_copies(k):
            cp.start()
    for k in (_NCHUNK - 2, _NCHUNK - 1):
        for cp in out_copies(k):
            cp.wait()


def kernel(input_ids, pos_embed):
    batch, seq_len = input_ids.shape
    d = pos_embed.shape[1]
    mesh = plsc.VectorSubcoreMesh(core_axis_name="c", subcore_axis_name="s")
    sc_call = pl.kernel(
        _sc_body,
        out_type=jax.ShapeDtypeStruct((batch, seq_len, d), jnp.float32),
        mesh=mesh,
        scratch_types=[
            pltpu.VMEM((_CHUNK, _D), jnp.float32),
            pltpu.VMEM((_CHUNK, _D), jnp.float32),
            pltpu.SemaphoreType.DMA,
            pltpu.SemaphoreType.DMA,
            pltpu.SemaphoreType.DMA,
            pltpu.SemaphoreType.DMA,
        ],
    )
    return sc_call(pos_embed[:seq_len])
